# baseline (speedup 1.0000x reference)
import functools

import jax
import jax.numpy as jnp
from jax import lax
from jax.experimental import pallas as pl
from jax.experimental.pallas import tpu as pltpu
from jax.experimental.pallas import tpu_sc as plsc

_NUM_CLASSES = 21
_BG = 20
_RATIO = 3
_N = 131072
_C = 8192

_NW_SC = 32
_CHUNK = _N // _NW_SC
_L = 16


def _sc_count_body(gt_hbm, cnt_hbm, gt_v, cnt_v):
    wid = lax.axis_index("s") * 2 + lax.axis_index("c")
    pltpu.sync_copy(gt_hbm.at[pl.ds(wid * _CHUNK, _CHUNK)], gt_v)

    def body(i, acc):
        v = gt_v[pl.ds(i * _L, _L)]
        return acc + jnp.where(v == _BG, 1, 0)

    acc = lax.fori_loop(0, _CHUNK // _L, body, jnp.zeros((_L,), jnp.int32))
    cnt_v[...] = acc
    pltpu.sync_copy(cnt_v, cnt_hbm.at[wid])


def _sc_sel_body(gt_hbm, cnt_hbm, sel_hbm, gt_v, cnt_v, sel_v):
    wid = lax.axis_index("s") * 2 + lax.axis_index("c")
    base = wid * _CHUNK
    pltpu.sync_copy(gt_hbm.at[pl.ds(base, _CHUNK)], gt_v)
    pltpu.sync_copy(cnt_hbm, cnt_v)

    def cbody(w, carry):
        pref, tot = carry
        cw = jnp.sum(cnt_v[w])
        return (pref + jnp.where(w < wid, cw, 0), tot + cw)

    pref, tot = lax.fori_loop(0, _NW_SC, cbody,
                              (jnp.int32(0), jnp.int32(0)))
    k = _RATIO * (_N - tot)

    def sbody(i, rank):
        v = gt_v[pl.ds(i * _L, _L)]
        neg = v == _BG
        ones = jnp.where(neg, 1, 0)
        incl = jnp.cumsum(ones)
        r = rank + incl - 1
        sel_v[pl.ds(i * _L, _L)] = jnp.where(neg & (r < k), 1.0, 0.0)
        return rank + jnp.sum(ones)

    lax.fori_loop(0, _CHUNK // _L, sbody, pref)
    pltpu.sync_copy(sel_v, sel_hbm.at[pl.ds(base, _CHUNK)])


def _sc_selection(gt):
    mesh = plsc.VectorSubcoreMesh(core_axis_name="c", subcore_axis_name="s")
    cnt = pl.kernel(
        _sc_count_body,
        mesh=mesh,
        compiler_params=pltpu.CompilerParams(needs_layout_passes=False),
        out_type=jax.ShapeDtypeStruct((_NW_SC, _L), jnp.int32),
        scratch_types=[
            pltpu.VMEM((_CHUNK,), jnp.int32),
            pltpu.VMEM((_L,), jnp.int32),
        ],
    )(gt)
    return pl.kernel(
        _sc_sel_body,
        mesh=mesh,
        compiler_params=pltpu.CompilerParams(needs_layout_passes=False),
        out_type=jax.ShapeDtypeStruct((_N,), jnp.float32),
        scratch_types=[
            pltpu.VMEM((_CHUNK,), jnp.int32),
            pltpu.VMEM((_NW_SC, _L), jnp.int32),
            pltpu.VMEM((_CHUNK,), jnp.float32),
        ],
    )(gt, cnt)


def _tc_body(cats_ref, bbs_ref, gtb_ref, gt_ref, sel_ref, out_ref, acc_ref):
    j = pl.program_id(0)

    @pl.when(j == 0)
    def _init():
        acc_ref[0] = 0.0
        acc_ref[1] = 0.0
        acc_ref[2] = 0.0
        acc_ref[3] = 0.0

    x = cats_ref[...]
    gt = gt_ref[...]
    sel = sel_ref[...]
    posf = jnp.where(gt != _BG, 1.0, 0.0)

    sp = jnp.maximum(x, 0.0) + jnp.log1p(jnp.exp(-jnp.abs(x)))
    row = lax.broadcasted_iota(jnp.int32, x.shape, 0)
    w = posf + sel
    conf_part = jnp.sum(jnp.where(row < _BG, sp, 0.0) * w)
    xc_part = jnp.sum(jnp.where(row == gt, x, 0.0) * posf)

    d = bbs_ref[...] - gtb_ref[...]
    ad = jnp.abs(d)
    l1 = jnp.where(ad < 1.0, 0.5 * d * d, ad - 0.5)
    loc_part = jnp.sum(l1 * posf)
    np_part = jnp.sum(posf)

    acc_ref[0] += np_part
    acc_ref[1] += conf_part - xc_part
    acc_ref[2] += loc_part
    acc_ref[3] += 0.0

    @pl.when(j == pl.num_programs(0) - 1)
    def _fini():
        n = acc_ref[0]
        conf = acc_ref[1]
        loc = acc_ref[2]
        out_ref[0] = (conf + loc) / n
        out_ref[1] = loc
        out_ref[2] = conf


def _tc_loss(catsT, bbsT, gtbT, gt1, sel1):
    return pl.pallas_call(
        _tc_body,
        grid=(_N // _C,),
        in_specs=[
            pl.BlockSpec((_NUM_CLASSES, _C), lambda j: (0, j)),
            pl.BlockSpec((4, _C), lambda j: (0, j)),
            pl.BlockSpec((4, _C), lambda j: (0, j)),
            pl.BlockSpec((1, _C), lambda j: (0, j)),
            pl.BlockSpec((1, _C), lambda j: (0, j)),
        ],
        out_specs=pl.BlockSpec(memory_space=pltpu.SMEM),
        out_shape=jax.ShapeDtypeStruct((3,), jnp.float32),
        scratch_shapes=[pltpu.SMEM((4,), jnp.float32)],
    )(catsT, bbsT, gtbT, gt1, sel1)


def kernel(bbs_preds, cats_preds, gt_bbs, gt_cats):
    gt = gt_cats.astype(jnp.int32)
    sel = _sc_selection(gt)
    out = _tc_loss(
        cats_preds.T,
        bbs_preds.T,
        gt_bbs.T,
        gt.reshape(1, _N),
        sel.reshape(1, _N),
    )
    return (out[0], out[1], out[2])
